# initial kernel scaffold (unmeasured)
import jax
import jax.numpy as jnp
from jax import lax
from jax.experimental import pallas as pl
from jax.experimental.pallas import tpu as pltpu

N_DEV = 8
B, SQ, D_MODEL, HQ, DH = 2, 256, 512, 4, 64
BLK = 64


def kernel(x, Wq, K_ext, V_ext, Wo):
    K2 = K_ext.reshape(B, SQ, HQ * DH)
    V2 = V_ext.reshape(B, SQ, HQ * DH)

    def body(x_ref, wq_ref, k_ref, v_ref, wo_ref, out_ref, send_sem, recv_sem):
        my_i = lax.axis_index("i")

        @pl.when(my_i == 0)
        def _compute():
            wq = wq_ref[...]
            wo = wo_ref[...]
            q_blk = lax.broadcasted_iota(jnp.int32, (SQ, SQ), 0) // BLK
            k_blk = lax.broadcasted_iota(jnp.int32, (SQ, SQ), 1) // BLK
            mask = k_blk <= q_blk
            for b in range(B):
                q = jnp.dot(x_ref[b], wq)
                out_b = jnp.zeros((SQ, D_MODEL), jnp.float32)
                for h in range(HQ):
                    q_h = q[:, h * DH:(h + 1) * DH]
                    k_h = k_ref[b, :, h * DH:(h + 1) * DH]
                    v_h = v_ref[b, :, h * DH:(h + 1) * DH]
                    s = lax.dot_general(
                        q_h, k_h, (((1,), (1,)), ((), ()))
                    ) * 0.125
                    s = jnp.where(mask, s, -1e9)
                    m = jnp.max(s, axis=-1, keepdims=True)
                    w = jnp.exp(s - m)
                    w = w / jnp.sum(w, axis=-1, keepdims=True)
                    ctx = jnp.dot(w, v_h)
                    out_b = out_b + jnp.dot(ctx, wo[h * DH:(h + 1) * DH, :])
                out_ref[b] = out_b

        @pl.when(my_i > 0)
        def _recv():
            recv = pltpu.make_async_remote_copy(
                src_ref=out_ref, dst_ref=out_ref,
                send_sem=send_sem, recv_sem=recv_sem,
                device_id=(my_i - 1,), device_id_type=pl.DeviceIdType.MESH,
            )
            recv.wait_recv()

        @pl.when(my_i < N_DEV - 1)
        def _send():
            send = pltpu.make_async_remote_copy(
                src_ref=out_ref, dst_ref=out_ref,
                send_sem=send_sem, recv_sem=recv_sem,
                device_id=(my_i + 1,), device_id_type=pl.DeviceIdType.MESH,
            )
            send.start()
            send.wait_send()

    out_shape = jax.ShapeDtypeStruct((B, SQ, D_MODEL), jnp.float32)
    return pl.pallas_call(
        body,
        out_shape=out_shape,
        in_specs=[pl.BlockSpec(memory_space=pltpu.VMEM)] * 5,
        out_specs=pl.BlockSpec(memory_space=pltpu.VMEM),
        scratch_shapes=[
            pltpu.SemaphoreType.DMA,
            pltpu.SemaphoreType.DMA,
        ],
        compiler_params=pltpu.CompilerParams(collective_id=0),
    )(x, Wq, K2, V2, Wo)


# baseline (device time: 62185 ns/iter reference)
import jax
import jax.numpy as jnp
from jax import lax
from jax.experimental import pallas as pl
from jax.experimental.pallas import tpu as pltpu

N_DEV = 8
B, SQ, D_MODEL, HQ, DH = 2, 256, 512, 4, 64
BLK = 64


def kernel(x, Wq, K_ext, V_ext, Wo):
    K2 = K_ext.reshape(B, SQ, HQ * DH)
    V2 = V_ext.reshape(B, SQ, HQ * DH)

    def body(x_ref, wq_ref, k_ref, v_ref, wo_ref, out_ref, send_sem, recv_sem):
        my_i = lax.axis_index("i")

        @pl.when(my_i == 0)
        def _compute():
            wq = wq_ref[...]
            wo = wo_ref[...]
            q_blk = lax.broadcasted_iota(jnp.int32, (SQ, SQ), 0) // BLK
            k_blk = lax.broadcasted_iota(jnp.int32, (SQ, SQ), 1) // BLK
            mask = k_blk <= q_blk
            for b in range(B):
                q = jnp.dot(x_ref[b], wq)
                out_b = jnp.zeros((SQ, D_MODEL), jnp.float32)
                for h in range(HQ):
                    q_h = q[:, h * DH:(h + 1) * DH]
                    k_h = k_ref[b, :, h * DH:(h + 1) * DH]
                    v_h = v_ref[b, :, h * DH:(h + 1) * DH]
                    s = lax.dot_general(
                        q_h, k_h, (((1,), (1,)), ((), ()))
                    ) * 0.125
                    s = jnp.where(mask, s, -1e9)
                    m = jnp.max(s, axis=-1, keepdims=True)
                    w = jnp.exp(s - m)
                    w = w / jnp.sum(w, axis=-1, keepdims=True)
                    ctx = jnp.dot(w, v_h)
                    out_b = out_b + jnp.dot(ctx, wo[h * DH:(h + 1) * DH, :])
                out_ref[b] = out_b

        @pl.when(my_i > 0)
        def _recv():
            recv = pltpu.make_async_remote_copy(
                src_ref=out_ref, dst_ref=out_ref,
                send_sem=send_sem, recv_sem=recv_sem,
                device_id=(my_i - 1,), device_id_type=pl.DeviceIdType.MESH,
            )
            recv.wait_recv()

        @pl.when(my_i < N_DEV - 1)
        def _send():
            send = pltpu.make_async_remote_copy(
                src_ref=out_ref, dst_ref=out_ref,
                send_sem=send_sem, recv_sem=recv_sem,
                device_id=(my_i + 1,), device_id_type=pl.DeviceIdType.MESH,
            )
            send.start()
            send.wait_send()

    out_shape = jax.ShapeDtypeStruct((B, SQ, D_MODEL), jnp.float32)
    return pl.pallas_call(
        body,
        out_shape=out_shape,
        in_specs=[pl.BlockSpec(memory_space=pltpu.VMEM)] * 5,
        out_specs=pl.BlockSpec(memory_space=pltpu.VMEM),
        scratch_shapes=[
            pltpu.SemaphoreType.DMA,
            pltpu.SemaphoreType.DMA,
        ],
    )(x, Wq, K2, V2, Wo)


# device time: 25447 ns/iter; 2.4437x vs baseline; 2.4437x over previous
import jax
import jax.numpy as jnp
from jax import lax
from jax.experimental import pallas as pl
from jax.experimental.pallas import tpu as pltpu

N_DEV = 8
B, SQ, D_MODEL, HQ, DH = 2, 256, 512, 4, 64
BLK = 64
NCHUNK = B * HQ
NEDGE = 3
NONE = N_DEV


def kernel(x, Wq, K_ext, V_ext, Wo):
    K2 = K_ext.reshape(B, SQ, HQ * DH)
    V2 = V_ext.reshape(B, SQ, HQ * DH)

    def body(x_ref, wq_ref, k_ref, v_ref, wo_ref, out_ref,
             ctx_ref, q_ref, send_sems, recv_sems):
        my_i = lax.axis_index("i")

        t0 = jnp.where(my_i == 0, 4,
             jnp.where(my_i == 1, 2,
             jnp.where(my_i == 4, 5,
             jnp.where(my_i == 5, 6, NONE))))
        t1 = jnp.where(my_i == 0, 1, jnp.where(my_i == 4, 7, NONE))
        t2 = jnp.where(my_i == 0, 3, NONE)
        tgts = [t0, t1, t2]

        @pl.when(my_i == 0)
        def _qproj():
            for b in range(B):
                q_ref[b] = jnp.dot(x_ref[b], wq_ref[...])

        q_blk = lax.broadcasted_iota(jnp.int32, (SQ, SQ), 0) // BLK
        k_blk = lax.broadcasted_iota(jnp.int32, (SQ, SQ), 1) // BLK
        mask = k_blk <= q_blk

        for c in range(NCHUNK):
            b, h = divmod(c, HQ)

            @pl.when(my_i == 0)
            def _compute(b=b, h=h):
                q_h = q_ref[b, :, h * DH:(h + 1) * DH]
                k_h = k_ref[b, :, h * DH:(h + 1) * DH]
                v_h = v_ref[b, :, h * DH:(h + 1) * DH]
                s = lax.dot_general(
                    q_h, k_h, (((1,), (1,)), ((), ()))
                ) * 0.125
                s = jnp.where(mask, s, -1e9)
                m = jnp.max(s, axis=-1, keepdims=True)
                w = jnp.exp(s - m)
                w = w / jnp.sum(w, axis=-1, keepdims=True)
                ctx_ref[b, h] = jnp.dot(w, v_h)

            @pl.when(my_i > 0)
            def _recv(c=c, b=b, h=h):
                pltpu.make_async_remote_copy(
                    src_ref=ctx_ref.at[b, h], dst_ref=ctx_ref.at[b, h],
                    send_sem=send_sems.at[0, c], recv_sem=recv_sems.at[c],
                    device_id=(0,), device_id_type=pl.DeviceIdType.MESH,
                ).wait_recv()

            for e in range(NEDGE):
                @pl.when(tgts[e] < N_DEV)
                def _send(t=tgts[e], e=e, c=c, b=b, h=h):
                    pltpu.make_async_remote_copy(
                        src_ref=ctx_ref.at[b, h], dst_ref=ctx_ref.at[b, h],
                        send_sem=send_sems.at[e, c], recv_sem=recv_sems.at[c],
                        device_id=(t,), device_id_type=pl.DeviceIdType.MESH,
                    ).start()

        wo = wo_ref[...]
        for b in range(B):
            out_b = jnp.zeros((SQ, D_MODEL), jnp.float32)
            for h in range(HQ):
                out_b = out_b + jnp.dot(
                    ctx_ref[b, h], wo[h * DH:(h + 1) * DH, :]
                )
            out_ref[b] = out_b

        for c in range(NCHUNK):
            b, h = divmod(c, HQ)
            for e in range(NEDGE):
                @pl.when(tgts[e] < N_DEV)
                def _wait(t=tgts[e], e=e, c=c, b=b, h=h):
                    pltpu.make_async_remote_copy(
                        src_ref=ctx_ref.at[b, h], dst_ref=ctx_ref.at[b, h],
                        send_sem=send_sems.at[e, c], recv_sem=recv_sems.at[c],
                        device_id=(t,), device_id_type=pl.DeviceIdType.MESH,
                    ).wait_send()

    out_shape = jax.ShapeDtypeStruct((B, SQ, D_MODEL), jnp.float32)
    return pl.pallas_call(
        body,
        out_shape=out_shape,
        in_specs=[pl.BlockSpec(memory_space=pltpu.VMEM)] * 5,
        out_specs=pl.BlockSpec(memory_space=pltpu.VMEM),
        scratch_shapes=[
            pltpu.VMEM((B, HQ, SQ, DH), jnp.float32),
            pltpu.VMEM((B, SQ, HQ * DH), jnp.float32),
            pltpu.SemaphoreType.DMA((NEDGE, NCHUNK)),
            pltpu.SemaphoreType.DMA((NCHUNK,)),
        ],
    )(x, Wq, K2, V2, Wo)


# device time: 20703 ns/iter; 3.0037x vs baseline; 1.2291x over previous
import jax
import jax.numpy as jnp
from jax import lax
from jax.experimental import pallas as pl
from jax.experimental.pallas import tpu as pltpu

N_DEV = 8
B, SQ, D_MODEL, HQ, DH = 2, 256, 512, 4, 64
BLK = 64
NCHUNK = B * HQ
NEDGE = 4
NONE = N_DEV


def kernel(x, Wq, K_ext, V_ext, Wo):
    K2 = K_ext.reshape(B, SQ, HQ * DH)
    V2 = V_ext.reshape(B, SQ, HQ * DH)

    def body(x_ref, wq_ref, k_ref, v_ref, wo_ref, out_ref,
             ctx_ref, q_ref, send_sems, recv_sems):
        my_i = lax.axis_index("i")
        is_hub = (my_i == 0) | (my_i == 4)

        tgts = [
            jnp.where(my_i == 0, 4, jnp.where(my_i == 4, 5, NONE)),
            jnp.where(my_i == 0, 1, jnp.where(my_i == 4, 6, NONE)),
            jnp.where(my_i == 0, 2, jnp.where(my_i == 4, 7, NONE)),
            jnp.where(my_i == 0, 3, NONE),
        ]
        parent = jnp.where(my_i == 0, NONE, jnp.where(my_i <= 4, 0, 4))

        barrier = pltpu.get_barrier_semaphore()
        for e in range(NEDGE):
            @pl.when(tgts[e] < N_DEV)
            def _sig(t=tgts[e]):
                pl.semaphore_signal(
                    barrier, inc=1, device_id=(t,),
                    device_id_type=pl.DeviceIdType.MESH,
                )

        @pl.when(parent < N_DEV)
        def _sig_parent():
            pl.semaphore_signal(
                barrier, inc=1, device_id=(parent,),
                device_id_type=pl.DeviceIdType.MESH,
            )

        @pl.when(is_hub)
        def _wait_hub():
            pl.semaphore_wait(barrier, 4)

        @pl.when(~is_hub)
        def _wait_leaf():
            pl.semaphore_wait(barrier, 1)

        @pl.when(my_i == 0)
        def _qproj():
            for b in range(B):
                q_ref[b] = jnp.dot(x_ref[b], wq_ref[...])

        q_blk = lax.broadcasted_iota(jnp.int32, (SQ, SQ), 0) // BLK
        k_blk = lax.broadcasted_iota(jnp.int32, (SQ, SQ), 1) // BLK
        mask = k_blk <= q_blk

        for c in range(NCHUNK):
            b, h = divmod(c, HQ)

            @pl.when(my_i == 0)
            def _compute(b=b, h=h):
                q_h = q_ref[b, :, h * DH:(h + 1) * DH]
                k_h = k_ref[b, :, h * DH:(h + 1) * DH]
                v_h = v_ref[b, :, h * DH:(h + 1) * DH]
                s = lax.dot_general(
                    q_h, k_h, (((1,), (1,)), ((), ()))
                ) * 0.125
                s = jnp.where(mask, s, -1e9)
                m = jnp.max(s, axis=-1, keepdims=True)
                w = jnp.exp(s - m)
                w = w / jnp.sum(w, axis=-1, keepdims=True)
                ctx_ref[b, h] = jnp.dot(w, v_h).astype(jnp.bfloat16)

            @pl.when(my_i > 0)
            def _recv(c=c, b=b, h=h):
                pltpu.make_async_remote_copy(
                    src_ref=ctx_ref.at[b, h], dst_ref=ctx_ref.at[b, h],
                    send_sem=send_sems.at[0, c], recv_sem=recv_sems.at[c],
                    device_id=(0,), device_id_type=pl.DeviceIdType.MESH,
                ).wait_recv()

            for e in range(NEDGE):
                @pl.when(tgts[e] < N_DEV)
                def _send(t=tgts[e], e=e, c=c, b=b, h=h):
                    pltpu.make_async_remote_copy(
                        src_ref=ctx_ref.at[b, h], dst_ref=ctx_ref.at[b, h],
                        send_sem=send_sems.at[e, c], recv_sem=recv_sems.at[c],
                        device_id=(t,), device_id_type=pl.DeviceIdType.MESH,
                    ).start()

        wo = wo_ref[...].astype(jnp.bfloat16)
        for b in range(B):
            out_b = jnp.zeros((SQ, D_MODEL), jnp.float32)
            for h in range(HQ):
                out_b = out_b + jnp.dot(
                    ctx_ref[b, h], wo[h * DH:(h + 1) * DH, :],
                    preferred_element_type=jnp.float32,
                )
            out_ref[b] = out_b

        for c in range(NCHUNK):
            b, h = divmod(c, HQ)
            for e in range(NEDGE):
                @pl.when(tgts[e] < N_DEV)
                def _wait(t=tgts[e], e=e, c=c, b=b, h=h):
                    pltpu.make_async_remote_copy(
                        src_ref=ctx_ref.at[b, h], dst_ref=ctx_ref.at[b, h],
                        send_sem=send_sems.at[e, c], recv_sem=recv_sems.at[c],
                        device_id=(t,), device_id_type=pl.DeviceIdType.MESH,
                    ).wait_send()

    out_shape = jax.ShapeDtypeStruct((B, SQ, D_MODEL), jnp.float32)
    return pl.pallas_call(
        body,
        out_shape=out_shape,
        in_specs=[pl.BlockSpec(memory_space=pltpu.VMEM)] * 5,
        out_specs=pl.BlockSpec(memory_space=pltpu.VMEM),
        scratch_shapes=[
            pltpu.VMEM((B, HQ, SQ, DH), jnp.bfloat16),
            pltpu.VMEM((B, SQ, HQ * DH), jnp.float32),
            pltpu.SemaphoreType.DMA((NEDGE, NCHUNK)),
            pltpu.SemaphoreType.DMA((NCHUNK,)),
        ],
        compiler_params=pltpu.CompilerParams(collective_id=0),
    )(x, Wq, K2, V2, Wo)


# device time: 20571 ns/iter; 3.0229x vs baseline; 1.0064x over previous
import jax
import jax.numpy as jnp
from jax import lax
from jax.experimental import pallas as pl
from jax.experimental.pallas import tpu as pltpu

N_DEV = 8
B, SQ, D_MODEL, HQ, DH = 2, 256, 512, 4, 64
BLK = 64
NCHUNK = B * HQ
NEDGE = 4
NONE = N_DEV


def kernel(x, Wq, K_ext, V_ext, Wo):
    K2 = K_ext.reshape(B, SQ, HQ * DH)
    V2 = V_ext.reshape(B, SQ, HQ * DH)

    def body(x_ref, wq_ref, k_ref, v_ref, wo_ref, out_ref,
             ctx_ref, q_ref, send_sems, recv_sems):
        my_i = lax.axis_index("i")
        is_hub = (my_i == 0) | (my_i == 4)

        tgts = [
            jnp.where(my_i == 0, 4, jnp.where(my_i == 4, 5, NONE)),
            jnp.where(my_i == 0, 1, jnp.where(my_i == 4, 6, NONE)),
            jnp.where(my_i == 0, 2, jnp.where(my_i == 4, 7, NONE)),
            jnp.where(my_i == 0, 3, NONE),
        ]
        parent = jnp.where(my_i == 0, NONE, jnp.where(my_i <= 4, 0, 4))

        barrier = pltpu.get_barrier_semaphore()
        for e in range(NEDGE):
            @pl.when(tgts[e] < N_DEV)
            def _sig(t=tgts[e]):
                pl.semaphore_signal(
                    barrier, inc=1, device_id=(t,),
                    device_id_type=pl.DeviceIdType.MESH,
                )

        @pl.when(parent < N_DEV)
        def _sig_parent():
            pl.semaphore_signal(
                barrier, inc=1, device_id=(parent,),
                device_id_type=pl.DeviceIdType.MESH,
            )

        @pl.when(is_hub)
        def _wait_hub():
            pl.semaphore_wait(barrier, 4)

        @pl.when(~is_hub)
        def _wait_leaf():
            pl.semaphore_wait(barrier, 1)

        @pl.when(my_i == 0)
        def _qproj():
            for b in range(B):
                q_ref[b] = jnp.dot(x_ref[b], wq_ref[...])

        q_blk = lax.broadcasted_iota(jnp.int32, (SQ, SQ), 0) // BLK
        k_blk = lax.broadcasted_iota(jnp.int32, (SQ, SQ), 1) // BLK
        mask = k_blk <= q_blk
        wo = wo_ref[...].astype(jnp.bfloat16)

        for c in range(NCHUNK):
            b, h = divmod(c, HQ)

            @pl.when(my_i == 0)
            def _compute(b=b, h=h):
                q_h = q_ref[b, :, h * DH:(h + 1) * DH]
                k_h = k_ref[b, :, h * DH:(h + 1) * DH]
                v_h = v_ref[b, :, h * DH:(h + 1) * DH]
                s = lax.dot_general(
                    q_h, k_h, (((1,), (1,)), ((), ()))
                ) * 0.125
                w = jnp.exp(jnp.where(mask, s, -1e9))
                r = 1.0 / jnp.sum(w, axis=-1, keepdims=True)
                ctx = jnp.dot(w, v_h) * r
                ctx_ref[b, h] = ctx.astype(jnp.bfloat16)

            @pl.when(my_i > 0)
            def _recv(c=c, b=b, h=h):
                pltpu.make_async_remote_copy(
                    src_ref=ctx_ref.at[b, h], dst_ref=ctx_ref.at[b, h],
                    send_sem=send_sems.at[0, c], recv_sem=recv_sems.at[c],
                    device_id=(0,), device_id_type=pl.DeviceIdType.MESH,
                ).wait_recv()

            for e in range(NEDGE):
                @pl.when(tgts[e] < N_DEV)
                def _send(t=tgts[e], e=e, c=c, b=b, h=h):
                    pltpu.make_async_remote_copy(
                        src_ref=ctx_ref.at[b, h], dst_ref=ctx_ref.at[b, h],
                        send_sem=send_sems.at[e, c], recv_sem=recv_sems.at[c],
                        device_id=(t,), device_id_type=pl.DeviceIdType.MESH,
                    ).start()

            d = jnp.dot(
                ctx_ref[b, h], wo[h * DH:(h + 1) * DH, :],
                preferred_element_type=jnp.float32,
            )
            if h == 0:
                out_ref[b] = d
            else:
                out_ref[b] = out_ref[b] + d

        for c in range(NCHUNK):
            b, h = divmod(c, HQ)
            for e in range(NEDGE):
                @pl.when(tgts[e] < N_DEV)
                def _wait(t=tgts[e], e=e, c=c, b=b, h=h):
                    pltpu.make_async_remote_copy(
                        src_ref=ctx_ref.at[b, h], dst_ref=ctx_ref.at[b, h],
                        send_sem=send_sems.at[e, c], recv_sem=recv_sems.at[c],
                        device_id=(t,), device_id_type=pl.DeviceIdType.MESH,
                    ).wait_send()

    out_shape = jax.ShapeDtypeStruct((B, SQ, D_MODEL), jnp.float32)
    return pl.pallas_call(
        body,
        out_shape=out_shape,
        in_specs=[pl.BlockSpec(memory_space=pltpu.VMEM)] * 5,
        out_specs=pl.BlockSpec(memory_space=pltpu.VMEM),
        scratch_shapes=[
            pltpu.VMEM((B, HQ, SQ, DH), jnp.bfloat16),
            pltpu.VMEM((B, SQ, HQ * DH), jnp.float32),
            pltpu.SemaphoreType.DMA((NEDGE, NCHUNK)),
            pltpu.SemaphoreType.DMA((NCHUNK,)),
        ],
        compiler_params=pltpu.CompilerParams(collective_id=0),
    )(x, Wq, K2, V2, Wo)
